# baseline (speedup 1.0000x reference)
import functools

import jax
import jax.numpy as jnp
from jax import lax
from jax.experimental import pallas as pl
from jax.experimental.pallas import tpu as pltpu
from jax.experimental.pallas import tpu_sc as plsc

NUM_CORES = 2
NUM_SUBCORES = 16
CHUNK = 32
PACK_W = 128
SRC_SHIFT = 15
G = 64


NBUF = 8


def _make_agg(n_pad, n_chunks, orows):
  mesh = plsc.VectorSubcoreMesh(core_axis_name="c", subcore_axis_name="s")
  sp_rows = n_pad + 128
  assert n_chunks % NBUF == 0
  n_outer = n_chunks // NBUF

  cpr = PACK_W // CHUNK
  n_rows = n_chunks // cpr

  @functools.partial(
      pl.kernel,
      out_type=jax.ShapeDtypeStruct((2 * n_pad, 128), jnp.float32),
      mesh=mesh,
      scratch_types=[
          pltpu.VMEM((n_rows, PACK_W), jnp.int32),
          pltpu.VMEM((NBUF, CHUNK), jnp.int32),
          pltpu.VMEM((NBUF, CHUNK), jnp.int32),
          pltpu.VMEM((NBUF, CHUNK, 128), jnp.float32),
          pltpu.VMEM_SHARED((sp_rows, 128), jnp.float32),
      ] + [pltpu.SemaphoreType.DMA] * (2 * NBUF + 1),
  )
  def agg(h_hbm, pk_hbm, out_hbm, pk_v, src_v, dst_v, rows_v, acc, *sems):
    gsems = sems[:NBUF]
    ssems = sems[NBUF:2 * NBUF]
    isem = sems[2 * NBUF]
    c = lax.axis_index("c")
    s = lax.axis_index("s")
    c1 = pltpu.async_copy(pk_hbm.at[c * NUM_SUBCORES + s], pk_v, isem)
    c2 = pltpu.async_copy(h_hbm.at[pl.ds(c * n_pad + s * orows, orows)],
                          acc.at[pl.ds(s * orows, orows)], isem)
    c1.wait()
    c2.wait()
    plsc.subcore_barrier()

    def unpack(prow, pcol, b):
      for j in range(CHUNK // 16):
        v = pk_v[prow, pl.ds(pcol * CHUNK + 16 * j, 16)]
        src_v[b, pl.ds(16 * j, 16)] = lax.shift_right_logical(v, SRC_SHIFT)
        dst_v[b, pl.ds(16 * j, 16)] = lax.bitwise_and(v, (1 << SRC_SHIFT) - 1)

    for b in range(NBUF):
      unpack(b // cpr, b % cpr, b)
      pltpu.async_copy(h_hbm.at[src_v.at[b]], rows_v.at[b], gsems[b])

    @pl.loop(0, n_outer)
    def _(i):
      g0 = i * NBUF
      for b in range(NBUF):
        pltpu.make_async_copy(h_hbm.at[src_v.at[b]], rows_v.at[b],
                              gsems[b]).wait()
        pltpu.async_copy(rows_v.at[b], acc.at[dst_v.at[b]], ssems[b],
                         add=True)
      for b in range(NBUF):
        pltpu.make_async_copy(rows_v.at[b], acc.at[dst_v.at[b]],
                              ssems[b]).wait()
        @pl.when(g0 + NBUF + b < n_chunks)
        def _():
          gn = NBUF + b
          unpack(i * (NBUF // cpr) + gn // cpr, gn % cpr, b)
          pltpu.async_copy(h_hbm.at[src_v.at[b]], rows_v.at[b], gsems[b])

    plsc.subcore_barrier()
    pltpu.sync_copy(acc.at[pl.ds(s * orows, orows)],
                    out_hbm.at[pl.ds(c * n_pad + s * orows, orows)])

  return agg


def _mlp_body(m_ref, w1_ref, b1_ref, w2_ref, b2_ref, o_ref):
  bf = jnp.bfloat16
  m0 = m_ref[0].astype(bf)
  m1 = m_ref[1].astype(bf)
  w1 = w1_ref[...].astype(bf)
  y = (jnp.dot(m0, w1[0], preferred_element_type=jnp.float32)
       + jnp.dot(m1, w1[1], preferred_element_type=jnp.float32)
       + b1_ref[...])
  y = jnp.maximum(y, 0.0).astype(bf)
  z = (jnp.dot(y, w2_ref[...].astype(bf), preferred_element_type=jnp.float32)
       + b2_ref[...])
  z = jnp.maximum(z, 0.0)
  o_ref[0] = z[:, :128]
  o_ref[1] = z[:, 128:]


def _make_mlp(N, n_pad, bn):
  grid = (N // bn,)
  return pl.pallas_call(
      _mlp_body,
      grid=grid,
      in_specs=[
          pl.BlockSpec((2, bn, 128), lambda i: (0, i, 0)),
          pl.BlockSpec((2, 128, 256), lambda i: (0, 0, 0)),
          pl.BlockSpec((1, 256), lambda i: (0, 0)),
          pl.BlockSpec((256, 256), lambda i: (0, 0)),
          pl.BlockSpec((1, 256), lambda i: (0, 0)),
      ],
      out_specs=pl.BlockSpec((2, bn, 128), lambda i: (0, i, 0)),
      out_shape=jax.ShapeDtypeStruct((2, n_pad, 128), jnp.float32),
  )


def _pool_decode_body(m_ref, ew1_ref, eb1_ref, ew2_ref, eb2_ref,
                      batch_ref, w1_ref, b1_ref, w2_ref, b2_ref,
                      recon_ref, ge_ref, pooled_ref, counts_ref):
  bf = jnp.bfloat16
  p = pl.program_id(0)
  i = pl.program_id(1)
  bn = batch_ref.shape[0]
  sel = (batch_ref[...] == lax.broadcasted_iota(jnp.int32, (bn, G), 1))
  s_mat = sel.astype(bf)
  dims = (((0,), (0,)), ((), ()))

  @pl.when(jnp.logical_and(p == 0, i == 0))
  def _():
    pooled_ref[...] = jnp.zeros_like(pooled_ref)
    counts_ref[...] = jnp.zeros_like(counts_ref)

  @pl.when(p == 0)
  def _():
    ew1 = ew1_ref[...].astype(bf)
    y = (jnp.dot(m_ref[0].astype(bf), ew1[0], preferred_element_type=jnp.float32)
         + jnp.dot(m_ref[1].astype(bf), ew1[1],
                   preferred_element_type=jnp.float32)
         + eb1_ref[...])
    y = jnp.maximum(y, 0.0).astype(bf)
    h3 = (jnp.dot(y, ew2_ref[...].astype(bf), preferred_element_type=jnp.float32)
          + eb2_ref[...])
    h3 = jnp.maximum(h3, 0.0).astype(bf)
    pooled_ref[...] += lax.dot_general(s_mat, h3, dims,
                                       preferred_element_type=jnp.float32)
    counts_ref[...] += lax.dot_general(s_mat, jnp.ones((bn, 1), bf), dims,
                                       preferred_element_type=jnp.float32)
    recon_ref[...] = jnp.zeros_like(recon_ref)
    ge_ref[...] = jnp.zeros_like(ge_ref)

  @pl.when(p == 1)
  def _():
    inv = 1.0 / jnp.maximum(counts_ref[...], 1.0)
    ge = pooled_ref[...] * inv
    d1 = jnp.maximum(jnp.dot(ge, w1_ref[...],
                             preferred_element_type=jnp.float32)
                     + b1_ref[...], 0.0)
    d2 = (jnp.dot(d1, w2_ref[...], preferred_element_type=jnp.float32)
          + b2_ref[...])
    recon_ref[...] = jnp.dot(s_mat, d2.astype(bf),
                             preferred_element_type=jnp.float32)
    ge_ref[...] = ge


def _make_pool_decode(N, n_pad, D, Hmid, bn):
  nblk = N // bn
  return pl.pallas_call(
      _pool_decode_body,
      grid=(2, nblk),
      in_specs=[
          pl.BlockSpec((2, bn, 128), lambda p, i: (0, i * (1 - p), 0)),
          pl.BlockSpec((2, 128, 256), lambda p, i: (0, 0, 0)),
          pl.BlockSpec((1, 256), lambda p, i: (0, 0)),
          pl.BlockSpec((256, 256), lambda p, i: (0, 0)),
          pl.BlockSpec((1, 256), lambda p, i: (0, 0)),
          pl.BlockSpec((bn, 1), lambda p, i: (i, 0)),
          pl.BlockSpec((256, Hmid), lambda p, i: (0, 0)),
          pl.BlockSpec((1, Hmid), lambda p, i: (0, 0)),
          pl.BlockSpec((Hmid, D), lambda p, i: (0, 0)),
          pl.BlockSpec((1, D), lambda p, i: (0, 0)),
      ],
      out_specs=(
          pl.BlockSpec((bn, D), lambda p, i: (i, 0)),
          pl.BlockSpec((G, 256), lambda p, i: (0, 0)),
      ),
      out_shape=(
          jax.ShapeDtypeStruct((N, D), jnp.float32),
          jax.ShapeDtypeStruct((G, 256), jnp.float32),
      ),
      scratch_shapes=[
          pltpu.VMEM((G, 256), jnp.float32),
          pltpu.VMEM((G, 1), jnp.float32),
      ],
  )


@jax.jit
def kernel(x, edge_index, batch, enc_W1, enc_b1, enc_W2, enc_b2,
           dec_W1, dec_b1, dec_W2, dec_b2):
  N, D = x.shape
  E = edge_index.shape[1]
  L = enc_W1.shape[0]
  n_pad = ((N + 127) // 128) * 128
  orows = n_pad // NUM_SUBCORES
  sp_rows = n_pad + 128

  n_chunks = (E + NUM_SUBCORES * CHUNK - 1) // (NUM_SUBCORES * CHUNK)
  n_chunks = ((n_chunks + NBUF - 1) // NBUF) * NBUF
  per_tile = n_chunks * CHUNK
  e_pad = per_tile * NUM_SUBCORES

  src = edge_index[0]
  dst = edge_index[1]
  npad_e = e_pad - E
  pad_ids = jnp.arange(npad_e, dtype=jnp.int32)
  src_p = jnp.concatenate([src, pad_ids % N])
  dst_p = jnp.concatenate([dst, n_pad + pad_ids % (sp_rows - n_pad)])
  packed2 = jnp.stack([
      (src_p << SRC_SHIFT) | dst_p,
      ((src_p + n_pad) << SRC_SHIFT) | dst_p,
  ]).reshape(2 * NUM_SUBCORES, per_tile // PACK_W, PACK_W)

  x_pl = jnp.zeros((2, n_pad, 128), jnp.float32)
  x_pl = x_pl.at[:, :N, :].set(x.reshape(N, 2, 128).transpose(1, 0, 2))
  h = x_pl.reshape(2 * n_pad, 128)

  agg = _make_agg(n_pad, n_chunks, orows)
  mlp = _make_mlp(N, n_pad, 2000)

  for l in range(L - 1):
    m = agg(h, packed2)
    hn = mlp(m.reshape(2, n_pad, 128),
             enc_W1[l].reshape(2, 128, 256),
             enc_b1[l].reshape(1, 256),
             enc_W2[l],
             enc_b2[l].reshape(1, 256))
    h = hn.reshape(2 * n_pad, 128)

  m = agg(h, packed2)
  pool = _make_pool_decode(N, n_pad, D, dec_W1.shape[1], 2000)
  recon, ge = pool(m.reshape(2, n_pad, 128),
                   enc_W1[L - 1].reshape(2, 128, 256),
                   enc_b1[L - 1].reshape(1, 256),
                   enc_W2[L - 1],
                   enc_b2[L - 1].reshape(1, 256),
                   batch.reshape(N, 1),
                   dec_W1, dec_b1.reshape(1, dec_W1.shape[1]),
                   dec_W2, dec_b2.reshape(1, D))
  return recon, ge


# speedup vs baseline: 1.0059x; 1.0059x over previous
import functools

import jax
import jax.numpy as jnp
from jax import lax
from jax.experimental import pallas as pl
from jax.experimental.pallas import tpu as pltpu
from jax.experimental.pallas import tpu_sc as plsc

NUM_CORES = 2
NUM_SUBCORES = 16
CHUNK = 32
PACK_W = 128
SRC_SHIFT = 15
G = 64


NBUF = 8


def _make_agg(n_pad, n_chunks, orows):
  mesh = plsc.VectorSubcoreMesh(core_axis_name="c", subcore_axis_name="s")
  sp_rows = n_pad + 128
  assert n_chunks % NBUF == 0
  n_outer = n_chunks // NBUF

  cpr = PACK_W // CHUNK
  n_rows = n_chunks // cpr

  @functools.partial(
      pl.kernel,
      out_type=jax.ShapeDtypeStruct((2 * n_pad, 128), jnp.float32),
      mesh=mesh,
      scratch_types=[
          pltpu.VMEM((n_rows, PACK_W), jnp.int32),
          pltpu.VMEM((NBUF, CHUNK), jnp.int32),
          pltpu.VMEM((NBUF, CHUNK), jnp.int32),
          pltpu.VMEM((NBUF, CHUNK, 128), jnp.float32),
          pltpu.VMEM_SHARED((sp_rows, 128), jnp.float32),
      ] + [pltpu.SemaphoreType.DMA] * (2 * NBUF + 1),
  )
  def agg(h_hbm, pk_hbm, out_hbm, pk_v, src_v, dst_v, rows_v, acc, *sems):
    gsems = sems[:NBUF]
    ssems = sems[NBUF:2 * NBUF]
    isem = sems[2 * NBUF]
    c = lax.axis_index("c")
    s = lax.axis_index("s")
    c1 = pltpu.async_copy(pk_hbm.at[c * NUM_SUBCORES + s], pk_v, isem)
    c2 = pltpu.async_copy(h_hbm.at[pl.ds(c * n_pad + s * orows, orows)],
                          acc.at[pl.ds(s * orows, orows)], isem)
    c1.wait()
    c2.wait()
    plsc.subcore_barrier()

    def unpack(prow, pcol, b):
      for j in range(CHUNK // 16):
        v = pk_v[prow, pl.ds(pcol * CHUNK + 16 * j, 16)]
        src_v[b, pl.ds(16 * j, 16)] = lax.shift_right_logical(v, SRC_SHIFT)
        dst_v[b, pl.ds(16 * j, 16)] = lax.bitwise_and(v, (1 << SRC_SHIFT) - 1)

    for b in range(NBUF):
      unpack(b // cpr, b % cpr, b)
      pltpu.async_copy(h_hbm.at[src_v.at[b]], rows_v.at[b], gsems[b])

    @pl.loop(0, n_outer - 1)
    def _(i):
      for b in range(NBUF):
        pltpu.make_async_copy(h_hbm.at[src_v.at[b]], rows_v.at[b],
                              gsems[b]).wait()
        pltpu.async_copy(rows_v.at[b], acc.at[dst_v.at[b]], ssems[b],
                         add=True)
      for b in range(NBUF):
        pltpu.make_async_copy(rows_v.at[b], acc.at[dst_v.at[b]],
                              ssems[b]).wait()
        gn = NBUF + b
        unpack(i * (NBUF // cpr) + gn // cpr, gn % cpr, b)
        pltpu.async_copy(h_hbm.at[src_v.at[b]], rows_v.at[b], gsems[b])

    for b in range(NBUF):
      pltpu.make_async_copy(h_hbm.at[src_v.at[b]], rows_v.at[b],
                            gsems[b]).wait()
      pltpu.async_copy(rows_v.at[b], acc.at[dst_v.at[b]], ssems[b], add=True)
    for b in range(NBUF):
      pltpu.make_async_copy(rows_v.at[b], acc.at[dst_v.at[b]], ssems[b]).wait()

    plsc.subcore_barrier()
    pltpu.sync_copy(acc.at[pl.ds(s * orows, orows)],
                    out_hbm.at[pl.ds(c * n_pad + s * orows, orows)])

  return agg


def _mlp_body(m_ref, w1_ref, b1_ref, w2_ref, b2_ref, o_ref):
  bf = jnp.bfloat16
  m0 = m_ref[0].astype(bf)
  m1 = m_ref[1].astype(bf)
  w1 = w1_ref[...].astype(bf)
  y = (jnp.dot(m0, w1[0], preferred_element_type=jnp.float32)
       + jnp.dot(m1, w1[1], preferred_element_type=jnp.float32)
       + b1_ref[...])
  y = jnp.maximum(y, 0.0).astype(bf)
  z = (jnp.dot(y, w2_ref[...].astype(bf), preferred_element_type=jnp.float32)
       + b2_ref[...])
  z = jnp.maximum(z, 0.0)
  o_ref[0] = z[:, :128]
  o_ref[1] = z[:, 128:]


def _make_mlp(N, n_pad, bn):
  grid = (N // bn,)
  return pl.pallas_call(
      _mlp_body,
      grid=grid,
      in_specs=[
          pl.BlockSpec((2, bn, 128), lambda i: (0, i, 0)),
          pl.BlockSpec((2, 128, 256), lambda i: (0, 0, 0)),
          pl.BlockSpec((1, 256), lambda i: (0, 0)),
          pl.BlockSpec((256, 256), lambda i: (0, 0)),
          pl.BlockSpec((1, 256), lambda i: (0, 0)),
      ],
      out_specs=pl.BlockSpec((2, bn, 128), lambda i: (0, i, 0)),
      out_shape=jax.ShapeDtypeStruct((2, n_pad, 128), jnp.float32),
  )


def _pool_decode_body(m_ref, ew1_ref, eb1_ref, ew2_ref, eb2_ref,
                      batch_ref, w1_ref, b1_ref, w2_ref, b2_ref,
                      recon_ref, ge_ref):
  bf = jnp.bfloat16
  n = batch_ref.shape[0]
  ew1 = ew1_ref[...].astype(bf)
  y = (jnp.dot(m_ref[0].astype(bf), ew1[0], preferred_element_type=jnp.float32)
       + jnp.dot(m_ref[1].astype(bf), ew1[1], preferred_element_type=jnp.float32)
       + eb1_ref[...])
  y = jnp.maximum(y, 0.0).astype(bf)
  h3 = (jnp.dot(y, ew2_ref[...].astype(bf), preferred_element_type=jnp.float32)
        + eb2_ref[...])
  h3 = jnp.maximum(h3, 0.0).astype(bf)
  sel = (batch_ref[...] == lax.broadcasted_iota(jnp.int32, (n, G), 1))
  s_mat = sel.astype(bf)
  dims = (((0,), (0,)), ((), ()))
  pooled = lax.dot_general(s_mat, h3, dims, preferred_element_type=jnp.float32)
  counts = lax.dot_general(s_mat, jnp.ones((n, 1), bf), dims,
                           preferred_element_type=jnp.float32)
  inv = 1.0 / jnp.maximum(counts, 1.0)
  ge = pooled * inv
  d1 = jnp.maximum(jnp.dot(ge, w1_ref[...], preferred_element_type=jnp.float32)
                   + b1_ref[...], 0.0)
  d2 = jnp.dot(d1, w2_ref[...], preferred_element_type=jnp.float32) + b2_ref[...]
  recon_ref[...] = jnp.dot(s_mat, d2.astype(bf),
                           preferred_element_type=jnp.float32)
  ge_ref[...] = ge


def _make_pool_decode(N, n_pad, D, Hmid, bn):
  return pl.pallas_call(
      _pool_decode_body,
      grid=(1,),
      in_specs=[
          pl.BlockSpec((2, N, 128), lambda i: (0, 0, 0)),
          pl.BlockSpec((2, 128, 256), lambda i: (0, 0, 0)),
          pl.BlockSpec((1, 256), lambda i: (0, 0)),
          pl.BlockSpec((256, 256), lambda i: (0, 0)),
          pl.BlockSpec((1, 256), lambda i: (0, 0)),
          pl.BlockSpec((N, 1), lambda i: (0, 0)),
          pl.BlockSpec((256, Hmid), lambda i: (0, 0)),
          pl.BlockSpec((1, Hmid), lambda i: (0, 0)),
          pl.BlockSpec((Hmid, D), lambda i: (0, 0)),
          pl.BlockSpec((1, D), lambda i: (0, 0)),
      ],
      out_specs=(
          pl.BlockSpec((N, D), lambda i: (0, 0)),
          pl.BlockSpec((G, 256), lambda i: (0, 0)),
      ),
      out_shape=(
          jax.ShapeDtypeStruct((N, D), jnp.float32),
          jax.ShapeDtypeStruct((G, 256), jnp.float32),
      ),
  )


@jax.jit
def kernel(x, edge_index, batch, enc_W1, enc_b1, enc_W2, enc_b2,
           dec_W1, dec_b1, dec_W2, dec_b2):
  N, D = x.shape
  E = edge_index.shape[1]
  L = enc_W1.shape[0]
  n_pad = ((N + 127) // 128) * 128
  orows = n_pad // NUM_SUBCORES
  sp_rows = n_pad + 128

  n_chunks = (E + NUM_SUBCORES * CHUNK - 1) // (NUM_SUBCORES * CHUNK)
  n_chunks = ((n_chunks + NBUF - 1) // NBUF) * NBUF
  per_tile = n_chunks * CHUNK
  e_pad = per_tile * NUM_SUBCORES

  src = edge_index[0]
  dst = edge_index[1]
  npad_e = e_pad - E
  pad_ids = jnp.arange(npad_e, dtype=jnp.int32)
  src_p = jnp.concatenate([src, pad_ids % N])
  dst_p = jnp.concatenate([dst, n_pad + pad_ids % (sp_rows - n_pad)])
  packed2 = jnp.stack([
      (src_p << SRC_SHIFT) | dst_p,
      ((src_p + n_pad) << SRC_SHIFT) | dst_p,
  ]).reshape(2 * NUM_SUBCORES, per_tile // PACK_W, PACK_W)

  x_pl = jnp.zeros((2, n_pad, 128), jnp.float32)
  x_pl = x_pl.at[:, :N, :].set(x.reshape(N, 2, 128).transpose(1, 0, 2))
  h = x_pl.reshape(2 * n_pad, 128)

  agg = _make_agg(n_pad, n_chunks, orows)
  mlp = _make_mlp(N, n_pad, 2000)

  for l in range(L - 1):
    m = agg(h, packed2)
    hn = mlp(m.reshape(2, n_pad, 128),
             enc_W1[l].reshape(2, 128, 256),
             enc_b1[l].reshape(1, 256),
             enc_W2[l],
             enc_b2[l].reshape(1, 256))
    h = hn.reshape(2 * n_pad, 128)

  m = agg(h, packed2)
  pool = _make_pool_decode(N, n_pad, D, dec_W1.shape[1], 2000)
  recon, ge = pool(m.reshape(2, n_pad, 128),
                   enc_W1[L - 1].reshape(2, 128, 256),
                   enc_b1[L - 1].reshape(1, 256),
                   enc_W2[L - 1],
                   enc_b2[L - 1].reshape(1, 256),
                   batch.reshape(N, 1),
                   dec_W1, dec_b1.reshape(1, dec_W1.shape[1]),
                   dec_W2, dec_b2.reshape(1, D))
  return recon, ge


# speedup vs baseline: 1.0207x; 1.0147x over previous
import functools

import jax
import jax.numpy as jnp
from jax import lax
from jax.experimental import pallas as pl
from jax.experimental.pallas import tpu as pltpu
from jax.experimental.pallas import tpu_sc as plsc

NUM_CORES = 2
NUM_SUBCORES = 16
CHUNK = 32
PACK_W = 128
SRC_SHIFT = 15
G = 64


NBUF = 8


def _make_agg(n_pad, n_chunks, orows):
  mesh = plsc.VectorSubcoreMesh(core_axis_name="c", subcore_axis_name="s")
  sp_rows = n_pad + 128
  assert n_chunks % NBUF == 0
  n_outer = n_chunks // NBUF

  cpr = PACK_W // CHUNK
  n_rows = n_chunks // cpr

  @functools.partial(
      pl.kernel,
      out_type=jax.ShapeDtypeStruct((2 * n_pad, 128), jnp.float32),
      mesh=mesh,
      scratch_types=[
          pltpu.VMEM((n_rows, PACK_W), jnp.int32),
          pltpu.VMEM((NBUF, CHUNK), jnp.int32),
          pltpu.VMEM((NBUF, CHUNK), jnp.int32),
          pltpu.VMEM((NBUF, CHUNK, 128), jnp.float32),
          pltpu.VMEM_SHARED((sp_rows, 128), jnp.float32),
      ] + [pltpu.SemaphoreType.DMA] * (2 * NBUF + 2),
  )
  def agg(h_hbm, pk_hbm, out_hbm, pk_v, src_v, dst_v, rows_v, acc, *sems):
    gsems = sems[:NBUF]
    ssems = sems[NBUF:2 * NBUF]
    isem = sems[2 * NBUF]
    hsem = sems[2 * NBUF + 1]
    c = lax.axis_index("c")
    s = lax.axis_index("s")
    c1 = pltpu.async_copy(pk_hbm.at[c * NUM_SUBCORES + s], pk_v, isem)
    c2 = pltpu.async_copy(h_hbm.at[pl.ds(c * n_pad + s * orows, orows)],
                          acc.at[pl.ds(s * orows, orows)], hsem)
    c1.wait()

    def unpack(prow, pcol, b):
      for j in range(CHUNK // 16):
        v = pk_v[prow, pl.ds(pcol * CHUNK + 16 * j, 16)]
        src_v[b, pl.ds(16 * j, 16)] = lax.shift_right_logical(v, SRC_SHIFT)
        dst_v[b, pl.ds(16 * j, 16)] = lax.bitwise_and(v, (1 << SRC_SHIFT) - 1)

    for b in range(NBUF):
      unpack(b // cpr, b % cpr, b)
      pltpu.async_copy(h_hbm.at[src_v.at[b]], rows_v.at[b], gsems[b])

    c2.wait()
    plsc.subcore_barrier()

    @pl.loop(0, n_outer - 1)
    def _(i):
      for b in range(NBUF):
        pltpu.make_async_copy(h_hbm.at[src_v.at[b]], rows_v.at[b],
                              gsems[b]).wait()
        pltpu.async_copy(rows_v.at[b], acc.at[dst_v.at[b]], ssems[b],
                         add=True)
      for b in range(NBUF):
        pltpu.make_async_copy(rows_v.at[b], acc.at[dst_v.at[b]],
                              ssems[b]).wait()
        gn = NBUF + b
        unpack(i * (NBUF // cpr) + gn // cpr, gn % cpr, b)
        pltpu.async_copy(h_hbm.at[src_v.at[b]], rows_v.at[b], gsems[b])

    for b in range(NBUF):
      pltpu.make_async_copy(h_hbm.at[src_v.at[b]], rows_v.at[b],
                            gsems[b]).wait()
      pltpu.async_copy(rows_v.at[b], acc.at[dst_v.at[b]], ssems[b], add=True)
    for b in range(NBUF):
      pltpu.make_async_copy(rows_v.at[b], acc.at[dst_v.at[b]], ssems[b]).wait()

    plsc.subcore_barrier()
    pltpu.sync_copy(acc.at[pl.ds(s * orows, orows)],
                    out_hbm.at[pl.ds(c * n_pad + s * orows, orows)])

  return agg


def _mlp_body(m_ref, w1_ref, b1_ref, w2_ref, b2_ref, o_ref):
  bf = jnp.bfloat16
  m0 = m_ref[0].astype(bf)
  m1 = m_ref[1].astype(bf)
  w1 = w1_ref[...].astype(bf)
  y = (jnp.dot(m0, w1[0], preferred_element_type=jnp.float32)
       + jnp.dot(m1, w1[1], preferred_element_type=jnp.float32)
       + b1_ref[...])
  y = jnp.maximum(y, 0.0).astype(bf)
  z = (jnp.dot(y, w2_ref[...].astype(bf), preferred_element_type=jnp.float32)
       + b2_ref[...])
  z = jnp.maximum(z, 0.0)
  o_ref[0] = z[:, :128]
  o_ref[1] = z[:, 128:]


def _make_mlp(N, n_pad, bn):
  grid = (N // bn,)
  return pl.pallas_call(
      _mlp_body,
      grid=grid,
      in_specs=[
          pl.BlockSpec((2, bn, 128), lambda i: (0, i, 0)),
          pl.BlockSpec((2, 128, 256), lambda i: (0, 0, 0)),
          pl.BlockSpec((1, 256), lambda i: (0, 0)),
          pl.BlockSpec((256, 256), lambda i: (0, 0)),
          pl.BlockSpec((1, 256), lambda i: (0, 0)),
      ],
      out_specs=pl.BlockSpec((2, bn, 128), lambda i: (0, i, 0)),
      out_shape=jax.ShapeDtypeStruct((2, n_pad, 128), jnp.float32),
  )


def _pool_decode_body(m_ref, ew1_ref, eb1_ref, ew2_ref, eb2_ref,
                      batch_ref, w1_ref, b1_ref, w2_ref, b2_ref,
                      recon_ref, ge_ref):
  bf = jnp.bfloat16
  n = batch_ref.shape[0]
  ew1 = ew1_ref[...].astype(bf)
  y = (jnp.dot(m_ref[0].astype(bf), ew1[0], preferred_element_type=jnp.float32)
       + jnp.dot(m_ref[1].astype(bf), ew1[1], preferred_element_type=jnp.float32)
       + eb1_ref[...])
  y = jnp.maximum(y, 0.0).astype(bf)
  h3 = (jnp.dot(y, ew2_ref[...].astype(bf), preferred_element_type=jnp.float32)
        + eb2_ref[...])
  h3 = jnp.maximum(h3, 0.0).astype(bf)
  sel = (batch_ref[...] == lax.broadcasted_iota(jnp.int32, (n, G), 1))
  s_mat = sel.astype(bf)
  dims = (((0,), (0,)), ((), ()))
  pooled = lax.dot_general(s_mat, h3, dims, preferred_element_type=jnp.float32)
  counts = lax.dot_general(s_mat, jnp.ones((n, 1), bf), dims,
                           preferred_element_type=jnp.float32)
  inv = 1.0 / jnp.maximum(counts, 1.0)
  ge = pooled * inv
  d1 = jnp.maximum(jnp.dot(ge, w1_ref[...], preferred_element_type=jnp.float32)
                   + b1_ref[...], 0.0)
  d2 = jnp.dot(d1, w2_ref[...], preferred_element_type=jnp.float32) + b2_ref[...]
  recon_ref[...] = jnp.dot(s_mat, d2.astype(bf),
                           preferred_element_type=jnp.float32)
  ge_ref[...] = ge


def _make_pool_decode(N, n_pad, D, Hmid, bn):
  return pl.pallas_call(
      _pool_decode_body,
      grid=(1,),
      in_specs=[
          pl.BlockSpec((2, N, 128), lambda i: (0, 0, 0)),
          pl.BlockSpec((2, 128, 256), lambda i: (0, 0, 0)),
          pl.BlockSpec((1, 256), lambda i: (0, 0)),
          pl.BlockSpec((256, 256), lambda i: (0, 0)),
          pl.BlockSpec((1, 256), lambda i: (0, 0)),
          pl.BlockSpec((N, 1), lambda i: (0, 0)),
          pl.BlockSpec((256, Hmid), lambda i: (0, 0)),
          pl.BlockSpec((1, Hmid), lambda i: (0, 0)),
          pl.BlockSpec((Hmid, D), lambda i: (0, 0)),
          pl.BlockSpec((1, D), lambda i: (0, 0)),
      ],
      out_specs=(
          pl.BlockSpec((N, D), lambda i: (0, 0)),
          pl.BlockSpec((G, 256), lambda i: (0, 0)),
      ),
      out_shape=(
          jax.ShapeDtypeStruct((N, D), jnp.float32),
          jax.ShapeDtypeStruct((G, 256), jnp.float32),
      ),
  )


@jax.jit
def kernel(x, edge_index, batch, enc_W1, enc_b1, enc_W2, enc_b2,
           dec_W1, dec_b1, dec_W2, dec_b2):
  N, D = x.shape
  E = edge_index.shape[1]
  L = enc_W1.shape[0]
  n_pad = ((N + 127) // 128) * 128
  orows = n_pad // NUM_SUBCORES
  sp_rows = n_pad + 128

  n_chunks = (E + NUM_SUBCORES * CHUNK - 1) // (NUM_SUBCORES * CHUNK)
  n_chunks = ((n_chunks + NBUF - 1) // NBUF) * NBUF
  per_tile = n_chunks * CHUNK
  e_pad = per_tile * NUM_SUBCORES

  src = edge_index[0]
  dst = edge_index[1]
  npad_e = e_pad - E
  pad_ids = jnp.arange(npad_e, dtype=jnp.int32)
  src_p = jnp.concatenate([src, pad_ids % N])
  dst_p = jnp.concatenate([dst, n_pad + pad_ids % (sp_rows - n_pad)])
  packed2 = jnp.stack([
      (src_p << SRC_SHIFT) | dst_p,
      ((src_p + n_pad) << SRC_SHIFT) | dst_p,
  ]).reshape(2 * NUM_SUBCORES, per_tile // PACK_W, PACK_W)

  x_pl = jnp.zeros((2, n_pad, 128), jnp.float32)
  x_pl = x_pl.at[:, :N, :].set(x.reshape(N, 2, 128).transpose(1, 0, 2))
  h = x_pl.reshape(2 * n_pad, 128)

  agg = _make_agg(n_pad, n_chunks, orows)
  mlp = _make_mlp(N, n_pad, 2000)

  for l in range(L - 1):
    m = agg(h, packed2)
    hn = mlp(m.reshape(2, n_pad, 128),
             enc_W1[l].reshape(2, 128, 256),
             enc_b1[l].reshape(1, 256),
             enc_W2[l],
             enc_b2[l].reshape(1, 256))
    h = hn.reshape(2 * n_pad, 128)

  m = agg(h, packed2)
  pool = _make_pool_decode(N, n_pad, D, dec_W1.shape[1], 2000)
  recon, ge = pool(m.reshape(2, n_pad, 128),
                   enc_W1[L - 1].reshape(2, 128, 256),
                   enc_b1[L - 1].reshape(1, 256),
                   enc_W2[L - 1],
                   enc_b2[L - 1].reshape(1, 256),
                   batch.reshape(N, 1),
                   dec_W1, dec_b1.reshape(1, dec_W1.shape[1]),
                   dec_W2, dec_b2.reshape(1, D))
  return recon, ge
